# baseline (speedup 1.0000x reference)
import functools

import jax
import jax.numpy as jnp
from jax import lax
from jax.experimental import pallas as pl
from jax.experimental.pallas import tpu as pltpu
from jax.experimental.pallas import tpu_sc as plsc

D = 128
E = 320000
NC = 2
NS = 16
NW = NC * NS
E_W = E // NW
E_BLK = 400
N_BLK = E_W // E_BLK


def _dot_body(z_hbm, u_hbm, v_hbm, out_hbm,
              uidx_v, vidx_v, zu_v, zv_v, out_v, sem_u, sem_v):
    wid = lax.axis_index("s") * NC + lax.axis_index("c")
    base = wid * E_W

    def block(b, carry):
        off = base + b * E_BLK
        pltpu.sync_copy(u_hbm.at[pl.ds(off, E_BLK)], uidx_v)
        pltpu.sync_copy(v_hbm.at[pl.ds(off, E_BLK)], vidx_v)
        cu = pltpu.async_copy(z_hbm.at[uidx_v], zu_v, sem_u)
        cv = pltpu.async_copy(z_hbm.at[vidx_v], zv_v, sem_v)
        cu.wait()
        cv.wait()

        lane = lax.iota(jnp.int32, 16)
        zero = jnp.zeros((16,), jnp.float32)

        def group(g, c):
            rows = g * 16 + lane

            def dstep(d, accs):
                a0, a1 = accs
                d0 = jnp.full((16,), d, jnp.int32)
                d1 = d0 + (D // 2)
                uu0 = plsc.load_gather(zu_v, [rows, d0])
                vv0 = plsc.load_gather(zv_v, [rows, d0])
                uu1 = plsc.load_gather(zu_v, [rows, d1])
                vv1 = plsc.load_gather(zv_v, [rows, d1])
                return (a0 + uu0 * vv0, a1 + uu1 * vv1)

            a0, a1 = lax.fori_loop(0, D // 2, dstep, (zero, zero), unroll=8)
            out_v[pl.ds(g * 16, 16)] = a0 + a1
            return c

        lax.fori_loop(0, E_BLK // 16, group, 0, unroll=False)
        pltpu.sync_copy(out_v, out_hbm.at[pl.ds(off, E_BLK)])
        return carry

    lax.fori_loop(0, N_BLK, block, 0, unroll=False)


@functools.partial(jax.jit, donate_argnums=())
def _dot_sc(z, u, v):
    mesh = plsc.VectorSubcoreMesh(core_axis_name="c", subcore_axis_name="s")
    return pl.kernel(
        _dot_body,
        mesh=mesh,
        compiler_params=pltpu.CompilerParams(needs_layout_passes=False),
        out_type=jax.ShapeDtypeStruct((E,), jnp.float32),
        scratch_types=[
            pltpu.VMEM((E_BLK,), jnp.int32),
            pltpu.VMEM((E_BLK,), jnp.int32),
            pltpu.VMEM((E_BLK, D), jnp.float32),
            pltpu.VMEM((E_BLK, D), jnp.float32),
            pltpu.VMEM((E_BLK,), jnp.float32),
            pltpu.SemaphoreType.DMA,
            pltpu.SemaphoreType.DMA,
        ],
    )(z, u, v)


def kernel(z, edge_index):
    u = edge_index[0].astype(jnp.int32)
    v = edge_index[1].astype(jnp.int32)
    return _dot_sc(z, u, v)


# speedup vs baseline: 3.4781x; 3.4781x over previous
import functools

import jax
import jax.numpy as jnp
from jax import lax
from jax.experimental import pallas as pl
from jax.experimental.pallas import tpu as pltpu
from jax.experimental.pallas import tpu_sc as plsc

D = 128
E = 320000
NC = 2
NS = 16
NW = NC * NS
E_W = E // NW
E_BLK = 80
N_BLK = E_W // E_BLK


def _dot_body(z_hbm, u_hbm, v_hbm, out_hbm,
              uidx_v, vidx_v, zu0, zv0, zu1, zv1, out_v, s0, s1):
    wid = lax.axis_index("s") * NC + lax.axis_index("c")
    base = wid * E_W
    pltpu.sync_copy(u_hbm.at[pl.ds(base, E_W)], uidx_v)
    pltpu.sync_copy(v_hbm.at[pl.ds(base, E_W)], vidx_v)

    def copies(b, zu, zv, sem):
        off = b * E_BLK
        cu = pltpu.make_async_copy(
            z_hbm.at[uidx_v.at[pl.ds(off, E_BLK)]], zu, sem)
        cv = pltpu.make_async_copy(
            z_hbm.at[vidx_v.at[pl.ds(off, E_BLK)]], zv, sem)
        return cu, cv

    def start(b, zu, zv, sem):
        cu, cv = copies(b, zu, zv, sem)
        cu.start()
        cv.start()

    def wait(b, zu, zv, sem):
        cu, cv = copies(b, zu, zv, sem)
        cu.wait()
        cv.wait()

    lane = lax.iota(jnp.int32, 16)

    def compute(b, zu, zv):
        def group(g, c):
            res = jnp.zeros((16,), jnp.float32)
            for j in range(16):
                e = g * 16 + j
                acc = zu[e, pl.ds(0, 16)] * zv[e, pl.ds(0, 16)]
                for ch in range(1, D // 16):
                    acc = acc + (zu[e, pl.ds(ch * 16, 16)]
                                 * zv[e, pl.ds(ch * 16, 16)])
                res = jnp.where(lane == j, jnp.sum(acc), res)
            out_v[pl.ds(b * E_BLK + g * 16, 16)] = res
            return c

        lax.fori_loop(0, E_BLK // 16, group, 0, unroll=True)

    start(0, zu0, zv0, s0)

    def pair(i, c):
        b0 = 2 * i
        start(b0 + 1, zu1, zv1, s1)
        wait(b0, zu0, zv0, s0)
        compute(b0, zu0, zv0)
        start(b0 + 2, zu0, zv0, s0)
        wait(b0 + 1, zu1, zv1, s1)
        compute(b0 + 1, zu1, zv1)
        return c

    lax.fori_loop(0, N_BLK // 2, pair, 0, unroll=False)
    wait(N_BLK - 1, zu0, zv0, s0)
    compute(N_BLK - 1, zu0, zv0)

    pltpu.sync_copy(out_v, out_hbm.at[pl.ds(base, E_W)])


@functools.partial(jax.jit, donate_argnums=())
def _dot_sc(z, u, v):
    mesh = plsc.VectorSubcoreMesh(core_axis_name="c", subcore_axis_name="s")
    return pl.kernel(
        _dot_body,
        mesh=mesh,
        compiler_params=pltpu.CompilerParams(needs_layout_passes=False),
        out_type=jax.ShapeDtypeStruct((E,), jnp.float32),
        scratch_types=[
            pltpu.VMEM((E_W,), jnp.int32),
            pltpu.VMEM((E_W,), jnp.int32),
            pltpu.VMEM((E_BLK, D), jnp.float32),
            pltpu.VMEM((E_BLK, D), jnp.float32),
            pltpu.VMEM((E_BLK, D), jnp.float32),
            pltpu.VMEM((E_BLK, D), jnp.float32),
            pltpu.VMEM((E_W,), jnp.float32),
            pltpu.SemaphoreType.DMA,
            pltpu.SemaphoreType.DMA,
        ],
    )(z, u, v)


def kernel(z, edge_index):
    u = edge_index[0].astype(jnp.int32)
    v = edge_index[1].astype(jnp.int32)
    return _dot_sc(z, u, v)
